# baseline (speedup 1.0000x reference)
import jax
import jax.numpy as jnp
from jax.experimental import pallas as pl
from jax.experimental.pallas import tpu as pltpu


def _copy_body(x_ref, o_ref, sem):
    f = x_ref.shape[0]
    copies = [
        pltpu.make_async_copy(x_ref.at[i], o_ref.at[:, i], sem.at[i])
        for i in range(f)
    ]
    for c in copies:
        c.start()
    for c in copies:
        c.wait()


def kernel(features):
    F, B, C, H, W = features.shape
    N = C * H * W
    LANES = 128
    rows = N // LANES
    x = jnp.reshape(features, (F, B, rows, LANES))

    out = pl.pallas_call(
        _copy_body,
        in_specs=[pl.BlockSpec(memory_space=pltpu.MemorySpace.HBM)],
        out_specs=pl.BlockSpec(memory_space=pltpu.MemorySpace.HBM),
        out_shape=jax.ShapeDtypeStruct((B, F, rows, LANES), features.dtype),
        scratch_shapes=[pltpu.SemaphoreType.DMA((F,))],
    )(x)
    return jnp.reshape(out, (B, F * C * H, W))


# speedup vs baseline: 1.0002x; 1.0002x over previous
import jax
import jax.numpy as jnp
from jax.experimental import pallas as pl
from jax.experimental.pallas import tpu as pltpu


def _copy_body(x_ref, o_ref, sem):
    f, b = x_ref.shape[0], x_ref.shape[1]
    copies = [
        pltpu.make_async_copy(x_ref.at[i, j], o_ref.at[j, i], sem.at[i, j])
        for i in range(f)
        for j in range(b)
    ]
    for c in copies:
        c.start()
    for c in copies:
        c.wait()


def kernel(features):
    F, B, C, H, W = features.shape
    N = C * H * W
    LANES = 128
    rows = N // LANES
    x = jnp.reshape(features, (F, B, rows, LANES))

    out = pl.pallas_call(
        _copy_body,
        in_specs=[pl.BlockSpec(memory_space=pltpu.MemorySpace.HBM)],
        out_specs=pl.BlockSpec(memory_space=pltpu.MemorySpace.HBM),
        out_shape=jax.ShapeDtypeStruct((B, F, rows, LANES), features.dtype),
        scratch_shapes=[pltpu.SemaphoreType.DMA((F, B))],
    )(x)
    return jnp.reshape(out, (B, F * C * H, W))


# speedup vs baseline: 5.8687x; 5.8676x over previous
import jax
import jax.numpy as jnp
from jax.experimental import pallas as pl
from jax.experimental.pallas import tpu as pltpu


def _copy_body(x_ref, o_ref):
    o_ref[...] = jnp.reshape(x_ref[...], o_ref.shape)


def kernel(features):
    F, B, C, H, W = features.shape
    N = C * H * W
    LANES = 128
    rows = N // LANES
    x = jnp.reshape(features, (F, B, rows, LANES))

    out = pl.pallas_call(
        _copy_body,
        grid=(B,),
        in_specs=[pl.BlockSpec((F, 1, rows, LANES), lambda b: (0, b, 0, 0))],
        out_specs=pl.BlockSpec((1, F, rows, LANES), lambda b: (b, 0, 0, 0)),
        out_shape=jax.ShapeDtypeStruct((B, F, rows, LANES), features.dtype),
    )(x)
    return jnp.reshape(out, (B, F * C * H, W))


# speedup vs baseline: 5.8912x; 1.0038x over previous
import jax
import jax.numpy as jnp
from jax.experimental import pallas as pl
from jax.experimental.pallas import tpu as pltpu


def _copy_body(x_ref, o_ref, buf, in_sem, out_sem):
    f, b = x_ref.shape[0], x_ref.shape[1]
    chunks = [(i, j) for i in range(f) for j in range(b)]
    ins = [
        pltpu.make_async_copy(x_ref.at[i, j], buf.at[c], in_sem.at[c])
        for c, (i, j) in enumerate(chunks)
    ]
    outs = [
        pltpu.make_async_copy(buf.at[c], o_ref.at[j, i], out_sem.at[c])
        for c, (i, j) in enumerate(chunks)
    ]
    for cp in ins:
        cp.start()
    for c in range(len(chunks)):
        ins[c].wait()
        outs[c].start()
    for cp in outs:
        cp.wait()


def kernel(features):
    F, B, C, H, W = features.shape
    N = C * H * W
    LANES = 128
    rows = N // LANES
    x = jnp.reshape(features, (F, B, rows, LANES))

    out = pl.pallas_call(
        _copy_body,
        in_specs=[pl.BlockSpec(memory_space=pltpu.MemorySpace.HBM)],
        out_specs=pl.BlockSpec(memory_space=pltpu.MemorySpace.HBM),
        out_shape=jax.ShapeDtypeStruct((B, F, rows, LANES), features.dtype),
        scratch_shapes=[
            pltpu.VMEM((F * B, rows, LANES), features.dtype),
            pltpu.SemaphoreType.DMA((F * B,)),
            pltpu.SemaphoreType.DMA((F * B,)),
        ],
    )(x)
    return jnp.reshape(out, (B, F * C * H, W))


# speedup vs baseline: 7.6681x; 1.3016x over previous
import jax
import jax.numpy as jnp
from jax.experimental import pallas as pl
from jax.experimental.pallas import tpu as pltpu


def _copy_body(x_ref, o_ref):
    o_ref[...] = jnp.reshape(x_ref[...], o_ref.shape)


def kernel(features):
    F, B, C, H, W = features.shape
    out = pl.pallas_call(
        _copy_body,
        grid=(B, F),
        in_specs=[pl.BlockSpec((1, 1, C, H, W), lambda b, i: (i, b, 0, 0, 0))],
        out_specs=pl.BlockSpec((1, C * H, W), lambda b, i: (b, i, 0)),
        out_shape=jax.ShapeDtypeStruct((B, F * C * H, W), features.dtype),
    )(features)
    return out


# speedup vs baseline: 7.7843x; 1.0151x over previous
import jax
import jax.numpy as jnp
from jax.experimental import pallas as pl
from jax.experimental.pallas import tpu as pltpu


def _copy_body(x_ref, o_ref):
    o_ref[...] = jnp.reshape(x_ref[...], o_ref.shape)


def kernel(features):
    F, B, C, H, W = features.shape
    out = pl.pallas_call(
        _copy_body,
        grid=(B,),
        in_specs=[pl.BlockSpec((F, 1, C, H, W), lambda b: (0, b, 0, 0, 0))],
        out_specs=pl.BlockSpec((1, F * C * H, W), lambda b: (b, 0, 0)),
        out_shape=jax.ShapeDtypeStruct((B, F * C * H, W), features.dtype),
    )(features)
    return out


# speedup vs baseline: 8.0892x; 1.0392x over previous
import jax
import jax.numpy as jnp
from jax.experimental import pallas as pl
from jax.experimental.pallas import tpu as pltpu

_K = 12
_D = 6


def _copy_body(x_ref, o_ref, buf, in_sem, out_sem):
    f, b = x_ref.shape[0], x_ref.shape[1]
    chunks = [(i, j) for j in range(b) for i in range(f)]
    nc = len(chunks)

    def start_in(c):
        i, j = chunks[c]
        pltpu.make_async_copy(x_ref.at[i, j], buf.at[c % _K], in_sem.at[c % _K]).start()

    def wait_in(c):
        i, j = chunks[c]
        pltpu.make_async_copy(x_ref.at[i, j], buf.at[c % _K], in_sem.at[c % _K]).wait()

    def start_out(c):
        i, j = chunks[c]
        pltpu.make_async_copy(buf.at[c % _K], o_ref.at[j, i], out_sem.at[c % _K]).start()

    def wait_out(c):
        i, j = chunks[c]
        pltpu.make_async_copy(buf.at[c % _K], o_ref.at[j, i], out_sem.at[c % _K]).wait()

    for c in range(min(_K, nc)):
        start_in(c)
    waited = [False] * nc
    for c in range(nc):
        wait_in(c)
        start_out(c)
        m = c + _K - _D
        if _K <= m < nc:
            wait_out(m - _K)
            waited[m - _K] = True
            start_in(m)
    for c in range(nc):
        if not waited[c]:
            wait_out(c)


def kernel(features):
    F, B, C, H, W = features.shape
    out = pl.pallas_call(
        _copy_body,
        in_specs=[pl.BlockSpec(memory_space=pltpu.MemorySpace.HBM)],
        out_specs=pl.BlockSpec(memory_space=pltpu.MemorySpace.HBM),
        out_shape=jax.ShapeDtypeStruct((B, F, C, H, W), features.dtype),
        scratch_shapes=[
            pltpu.VMEM((_K, C, H, W), features.dtype),
            pltpu.SemaphoreType.DMA((_K,)),
            pltpu.SemaphoreType.DMA((_K,)),
        ],
    )(features)
    return jnp.reshape(out, (B, F * C * H, W))
